# Initial kernel scaffold; baseline (speedup 1.0000x reference)
import jax
import jax.numpy as jnp
from jax.experimental import pallas as pl


def kernel(x, W):
    raise NotImplementedError("write your pallas kernel here")


# speedup vs baseline: 2.2354x; 2.2354x over previous
import jax
import jax.numpy as jnp
from jax.experimental import pallas as pl
from jax.experimental.pallas import tpu as pltpu
from functools import partial

HIDDEN = 768
NUM_EXPERTS = 64
TOP_K = 2


def _router_kernel(x_ref, w_ref, rw_ref, se_ref, bl_ref, uv_ref, gn_ref,
                   en_ref, usage_acc, ent_acc, *, n_tokens, n_steps):
    i = pl.program_id(0)

    @pl.when(i == 0)
    def _init():
        usage_acc[...] = jnp.zeros_like(usage_acc)
        ent_acc[...] = jnp.zeros_like(ent_acc)

    xblk = x_ref[...]
    w = w_ref[...]
    logits = jax.lax.dot_general(
        xblk, w, (((1,), (1,)), ((), ())),
        preferred_element_type=jnp.float32)

    e = logits.shape[-1]
    iota = jax.lax.broadcasted_iota(jnp.int32, logits.shape, 1)

    m1 = jnp.max(logits, axis=-1, keepdims=True)
    idx1 = jnp.min(jnp.where(logits == m1, iota, e), axis=-1, keepdims=True)
    masked = jnp.where(iota == idx1, -jnp.inf, logits)
    m2 = jnp.max(masked, axis=-1, keepdims=True)
    idx2 = jnp.min(jnp.where(masked == m2, iota, e), axis=-1, keepdims=True)

    s = jnp.exp(m2 - m1)
    w1 = 1.0 / (1.0 + s)
    w2 = s / (1.0 + s)
    rw_ref[...] = jnp.concatenate([w1, w2], axis=1)
    se_ref[...] = jnp.concatenate([idx1, idx2], axis=1)

    ex = jnp.exp(logits - m1)
    denom = jnp.sum(ex, axis=-1, keepdims=True)
    p = ex / denom
    usage_acc[...] += jnp.sum(p, axis=0, keepdims=True)
    ent_acc[...] += jnp.sum(-p * jnp.log(p + 1e-10)).reshape(1, 1)

    @pl.when(i == n_steps - 1)
    def _finalize():
        u = usage_acc[...] / n_tokens
        su = jnp.sum(u)
        mean = su / e
        var = jnp.sum((u - mean) ** 2) / (e - 1)
        uv_ref[...] = var.reshape(1, 1)
        bl_ref[...] = (var * e).reshape(1, 1)
        en_ref[...] = ent_acc[...] / n_tokens
        ut = u.reshape(e, 1)
        lt = (u < ut).astype(jnp.float32)
        eq = (u == ut).astype(jnp.float32)
        c_lt = jnp.sum(lt, axis=1, keepdims=True)
        c_eq = jnp.sum(eq, axis=1, keepdims=True)
        ranksum = jnp.sum(ut * (c_lt + (c_eq + 1.0) * 0.5))
        gn_ref[...] = (2.0 * ranksum / (e * su) - (e + 1.0) / e).reshape(1, 1)


@jax.jit
def kernel(x, W):
    b, s, h = x.shape
    n_tokens = b * s
    block = 1024
    n_steps = n_tokens // block
    xf = x.reshape(n_tokens, h)

    scalar_spec = pl.BlockSpec((1, 1), lambda i: (0, 0))
    out = pl.pallas_call(
        partial(_router_kernel, n_tokens=n_tokens, n_steps=n_steps),
        grid=(n_steps,),
        in_specs=[
            pl.BlockSpec((block, h), lambda i: (i, 0)),
            pl.BlockSpec((NUM_EXPERTS, h), lambda i: (0, 0)),
        ],
        out_specs=[
            pl.BlockSpec((block, TOP_K), lambda i: (i, 0)),
            pl.BlockSpec((block, TOP_K), lambda i: (i, 0)),
            scalar_spec, scalar_spec, scalar_spec, scalar_spec,
        ],
        out_shape=[
            jax.ShapeDtypeStruct((n_tokens, TOP_K), jnp.float32),
            jax.ShapeDtypeStruct((n_tokens, TOP_K), jnp.int32),
            jax.ShapeDtypeStruct((1, 1), jnp.float32),
            jax.ShapeDtypeStruct((1, 1), jnp.float32),
            jax.ShapeDtypeStruct((1, 1), jnp.float32),
            jax.ShapeDtypeStruct((1, 1), jnp.float32),
        ],
        scratch_shapes=[
            pltpu.VMEM((1, NUM_EXPERTS), jnp.float32),
            pltpu.VMEM((1, 1), jnp.float32),
        ],
    )(xf, W)

    rw, se, bl, uv, gn, en = out
    return (rw.reshape(b, s, TOP_K), se.reshape(b, s, TOP_K),
            bl[0, 0], uv[0, 0], gn[0, 0], en[0, 0])
